# Initial kernel scaffold; baseline (speedup 1.0000x reference)
import jax
import jax.numpy as jnp
from jax.experimental import pallas as pl


def kernel(x, edge_index, W1, b1, W2, b2):
    raise NotImplementedError("write your pallas kernel here")


# speedup vs baseline: 6.1252x; 6.1252x over previous
import functools

import jax
import jax.numpy as jnp
from jax import lax
from jax.experimental import pallas as pl
from jax.experimental.pallas import tpu as pltpu
from jax.experimental.pallas import tpu_sc as plsc

N_NODES = 10000
EMB = 256
HALF = 128
N_EDGES = 160000

N_SUB = 16
EDGES_PER_TILE = N_EDGES // N_SUB
CHUNK = 125
N_CHUNKS = EDGES_PER_TILE // CHUNK
ROWS_MAIN = 624
ROW_TAIL = N_NODES - N_SUB * ROWS_MAIN
ZROWS = 104
N_ZCHUNKS = ROWS_MAIN // ZROWS

MLP_BLK = 1000


def _mlp_body(x_ref, w1_ref, b1_ref, w2_ref, b2_ref, h0_ref, h1_ref):
    h = jnp.dot(x_ref[...], w1_ref[...], preferred_element_type=jnp.float32)
    h = jnp.maximum(h + b1_ref[...], 0.0)
    h = jnp.dot(h, w2_ref[...], preferred_element_type=jnp.float32)
    h = jnp.maximum(h + b2_ref[...], 0.0)
    h0_ref[...] = h[:, :HALF]
    h1_ref[...] = h[:, HALF:]


def _mlp(x, W1, b1, W2, b2):
    return pl.pallas_call(
        _mlp_body,
        grid=(N_NODES // MLP_BLK,),
        in_specs=[
            pl.BlockSpec((MLP_BLK, EMB), lambda i: (i, 0)),
            pl.BlockSpec((EMB, EMB), lambda i: (0, 0)),
            pl.BlockSpec((1, EMB), lambda i: (0, 0)),
            pl.BlockSpec((EMB, EMB), lambda i: (0, 0)),
            pl.BlockSpec((1, EMB), lambda i: (0, 0)),
        ],
        out_specs=[
            pl.BlockSpec((MLP_BLK, HALF), lambda i: (i, 0)),
            pl.BlockSpec((MLP_BLK, HALF), lambda i: (i, 0)),
        ],
        out_shape=[
            jax.ShapeDtypeStruct((N_NODES, HALF), jnp.float32),
            jax.ShapeDtypeStruct((N_NODES, HALF), jnp.float32),
        ],
    )(x, W1, b1, W2, b2)


def _sc_message_pass(h0, h1, src3, dst3):
    mesh = plsc.VectorSubcoreMesh(core_axis_name="c", subcore_axis_name="s")

    @functools.partial(
        pl.kernel,
        mesh=mesh,
        out_type=jax.ShapeDtypeStruct((N_NODES, EMB), jnp.float32),
        scratch_types=[
            pltpu.VMEM((N_CHUNKS, CHUNK), jnp.int32),
            pltpu.VMEM((N_CHUNKS, CHUNK), jnp.int32),
            pltpu.VMEM((CHUNK, HALF), jnp.float32),
            pltpu.VMEM((ZROWS, HALF), jnp.float32),
            pltpu.VMEM_SHARED((N_NODES, HALF), jnp.float32),
            pltpu.SemaphoreType.DMA,
        ],
    )
    def k(h0_hbm, h1_hbm, src_hbm, dst_hbm, out_hbm, src_v, dst_v, rows_v,
          zbuf, acc, sem):
        c = lax.axis_index("c")
        s = lax.axis_index("s")
        row0 = s * ROWS_MAIN

        pltpu.sync_copy(src_hbm.at[s], src_v)
        pltpu.sync_copy(dst_hbm.at[s], dst_v)

        zv = jnp.zeros((16,), jnp.float32)

        def zbody(i, carry):
            zbuf[i // 8, pl.ds((i % 8) * 16, 16)] = zv
            return carry

        lax.fori_loop(0, ZROWS * 8, zbody, 0)
        for r in range(N_ZCHUNKS):
            pltpu.sync_copy(zbuf, acc.at[pl.ds(row0 + r * ZROWS, ZROWS)])

        @pl.when(s == 0)
        def _():
            pltpu.sync_copy(zbuf.at[pl.ds(0, ROW_TAIL)],
                            acc.at[pl.ds(N_SUB * ROWS_MAIN, ROW_TAIL)])

        plsc.subcore_barrier()

        def run(h_ref):
            def body(j, carry):
                pltpu.async_copy(h_ref.at[src_v.at[j]], rows_v, sem).wait()
                pltpu.sync_copy(rows_v, acc.at[dst_v.at[j]], add=True)
                return carry

            lax.fori_loop(0, N_CHUNKS, body, 0)

        @pl.when(c == 0)
        def _():
            run(h0_hbm)

        @pl.when(c == 1)
        def _():
            run(h1_hbm)

        plsc.subcore_barrier()

        def writeout(col0):
            pltpu.sync_copy(
                acc.at[pl.ds(row0, ROWS_MAIN)],
                out_hbm.at[pl.ds(row0, ROWS_MAIN), pl.ds(col0, HALF)])

            @pl.when(s == 0)
            def _():
                tail0 = N_SUB * ROWS_MAIN
                pltpu.sync_copy(
                    acc.at[pl.ds(tail0, ROW_TAIL)],
                    out_hbm.at[pl.ds(tail0, ROW_TAIL), pl.ds(col0, HALF)])

        @pl.when(c == 0)
        def _():
            writeout(0)

        @pl.when(c == 1)
        def _():
            writeout(HALF)

    return k(h0, h1, src3, dst3)


def kernel(x, edge_index, W1, b1, W2, b2):
    ei = edge_index.astype(jnp.int32)
    src3 = ei[0].reshape(N_SUB, N_CHUNKS, CHUNK)
    dst3 = ei[1].reshape(N_SUB, N_CHUNKS, CHUNK)
    h0, h1 = _mlp(x, W1, b1.reshape(1, EMB), W2, b2.reshape(1, EMB))
    return _sc_message_pass(h0, h1, src3, dst3)


# speedup vs baseline: 8.8305x; 1.4417x over previous
import functools

import jax
import jax.numpy as jnp
from jax import lax
from jax.experimental import pallas as pl
from jax.experimental.pallas import tpu as pltpu
from jax.experimental.pallas import tpu_sc as plsc

N_NODES = 10000
EMB = 256
HALF = 128
N_EDGES = 160000

N_SUB = 16
EDGES_PER_TILE = N_EDGES // N_SUB
CHUNK = 125
N_CHUNKS = EDGES_PER_TILE // CHUNK
N_CH_HALF = N_CHUNKS // 2
ROWS_MAIN = 624
ROW_TAIL = N_NODES - N_SUB * ROWS_MAIN
ZROWS = 104
N_ZCHUNKS = ROWS_MAIN // ZROWS

MLP_BLK = 1000


def _mlp_body(x_ref, w1_ref, b1_ref, w2_ref, b2_ref, h0_ref, h1_ref):
    h = jnp.dot(x_ref[...], w1_ref[...], preferred_element_type=jnp.float32)
    h = jnp.maximum(h + b1_ref[...], 0.0)
    h = jnp.dot(h, w2_ref[...], preferred_element_type=jnp.float32)
    h = jnp.maximum(h + b2_ref[...], 0.0)
    h0_ref[...] = h[:, :HALF]
    h1_ref[...] = h[:, HALF:]


def _mlp(x, W1, b1, W2, b2):
    return pl.pallas_call(
        _mlp_body,
        grid=(N_NODES // MLP_BLK,),
        in_specs=[
            pl.BlockSpec((MLP_BLK, EMB), lambda i: (i, 0)),
            pl.BlockSpec((EMB, EMB), lambda i: (0, 0)),
            pl.BlockSpec((1, EMB), lambda i: (0, 0)),
            pl.BlockSpec((EMB, EMB), lambda i: (0, 0)),
            pl.BlockSpec((1, EMB), lambda i: (0, 0)),
        ],
        out_specs=[
            pl.BlockSpec((MLP_BLK, HALF), lambda i: (i, 0)),
            pl.BlockSpec((MLP_BLK, HALF), lambda i: (i, 0)),
        ],
        out_shape=[
            jax.ShapeDtypeStruct((N_NODES, HALF), jnp.float32),
            jax.ShapeDtypeStruct((N_NODES, HALF), jnp.float32),
        ],
    )(x, W1, b1, W2, b2)


def _sc_message_pass(h0, h1, src3, dst3):
    mesh = plsc.VectorSubcoreMesh(core_axis_name="c", subcore_axis_name="s")

    @functools.partial(
        pl.kernel,
        mesh=mesh,
        out_type=jax.ShapeDtypeStruct((N_NODES, EMB), jnp.float32),
        scratch_types=[
            pltpu.VMEM((N_CH_HALF, CHUNK), jnp.int32),
            pltpu.VMEM((N_CH_HALF, CHUNK), jnp.int32),
            pltpu.VMEM((CHUNK, HALF), jnp.float32),
            pltpu.VMEM((CHUNK, HALF), jnp.float32),
            pltpu.VMEM_SHARED((N_NODES, HALF), jnp.float32),
            pltpu.SemaphoreType.DMA,
            pltpu.SemaphoreType.DMA,
        ],
    )
    def k(h0_hbm, h1_hbm, src_hbm, dst_hbm, out_hbm, src_v, dst_v, rows_v,
          rows_b, acc, sem, sem_b):
        c = lax.axis_index("c")
        s = lax.axis_index("s")
        row0 = s * ROWS_MAIN

        zv = jnp.zeros((16,), jnp.float32)

        def zbody(i, carry):
            rows_v[i // 8, pl.ds((i % 8) * 16, 16)] = zv
            return carry

        lax.fori_loop(0, CHUNK * 8, zbody, 0)
        for r in range(N_ZCHUNKS):
            pltpu.sync_copy(rows_v.at[pl.ds(0, ZROWS)],
                            acc.at[pl.ds(row0 + r * ZROWS, ZROWS)])

        @pl.when(s == 0)
        def _():
            pltpu.sync_copy(rows_v.at[pl.ds(0, ROW_TAIL)],
                            acc.at[pl.ds(N_SUB * ROWS_MAIN, ROW_TAIL)])

        plsc.subcore_barrier()

        def run(h_ref):
            for half in range(2):
                pltpu.sync_copy(
                    src_hbm.at[s, pl.ds(half * N_CH_HALF, N_CH_HALF)], src_v)
                pltpu.sync_copy(
                    dst_hbm.at[s, pl.ds(half * N_CH_HALF, N_CH_HALF)], dst_v)
                pltpu.async_copy(h_ref.at[src_v.at[0]], rows_v, sem)

                def body(i, carry):
                    j0 = 2 * i
                    pltpu.async_copy(
                        h_ref.at[src_v.at[j0 + 1]], rows_b, sem_b)
                    pltpu.make_async_copy(
                        h_ref.at[src_v.at[j0]], rows_v, sem).wait()
                    pltpu.sync_copy(rows_v, acc.at[dst_v.at[j0]], add=True)

                    @pl.when(i < N_CH_HALF // 2 - 1)
                    def _():
                        pltpu.async_copy(
                            h_ref.at[src_v.at[j0 + 2]], rows_v, sem)

                    pltpu.make_async_copy(
                        h_ref.at[src_v.at[j0 + 1]], rows_b, sem_b).wait()
                    pltpu.sync_copy(
                        rows_b, acc.at[dst_v.at[j0 + 1]], add=True)
                    return carry

                lax.fori_loop(0, N_CH_HALF // 2, body, 0)

        @pl.when(c == 0)
        def _():
            run(h0_hbm)

        @pl.when(c == 1)
        def _():
            run(h1_hbm)

        plsc.subcore_barrier()

        def writeout(col0):
            pltpu.sync_copy(
                acc.at[pl.ds(row0, ROWS_MAIN)],
                out_hbm.at[pl.ds(row0, ROWS_MAIN), pl.ds(col0, HALF)])

            @pl.when(s == 0)
            def _():
                tail0 = N_SUB * ROWS_MAIN
                pltpu.sync_copy(
                    acc.at[pl.ds(tail0, ROW_TAIL)],
                    out_hbm.at[pl.ds(tail0, ROW_TAIL), pl.ds(col0, HALF)])

        @pl.when(c == 0)
        def _():
            writeout(0)

        @pl.when(c == 1)
        def _():
            writeout(HALF)

    return k(h0, h1, src3, dst3)


def kernel(x, edge_index, W1, b1, W2, b2):
    ei = edge_index.astype(jnp.int32)
    src3 = ei[0].reshape(N_SUB, N_CHUNKS, CHUNK)
    dst3 = ei[1].reshape(N_SUB, N_CHUNKS, CHUNK)
    h0, h1 = _mlp(x, W1, b1.reshape(1, EMB), W2, b2.reshape(1, EMB))
    return _sc_message_pass(h0, h1, src3, dst3)
